# baseline (speedup 1.0000x reference)
import functools

import jax
import jax.numpy as jnp
from jax import lax
from jax.experimental import pallas as pl
from jax.experimental.pallas import tpu as pltpu
from jax.experimental.pallas import tpu_sc as plsc

BATCH = 16384
DIM = 32
NW = 32
BPW = BATCH // NW
NCHUNK = 4
CHUNK = BPW // NCHUNK

NROWS = 1000000
RCH = 128
NCHT = NROWS // RCH + 1
TAIL = NROWS - (NCHT - 1) * RCH
KMAX = -(-NCHT // NW)


def _repack_body(Ut_h, It_h, Urm_h, Irm_h, stag, rows2, sem):
    cid = lax.axis_index("c")
    sid = lax.axis_index("s")
    wid = sid * 2 + cid

    lanes = lax.iota(jnp.int32, 16)

    def transpose_groups(ngrp):
        def g_body(g, _):
            row = g * 16 + lanes
            for d in range(DIM):
                vals = stag[d, pl.ds(g * 16, 16)]
                col = jnp.full((16,), d, jnp.int32)
                plsc.store_scatter(rows2, [row, col], vals)
            return 0

        lax.fori_loop(0, ngrp, g_body, 0)

    for src_h, dst_h in ((Ut_h, Urm_h), (It_h, Irm_h)):

        def k_body(k, _):
            c = wid + k * NW
            off = c * RCH

            @pl.when(c < NCHT - 1)
            def _full():
                pltpu.sync_copy(src_h.at[:, pl.ds(off, RCH)], stag)
                transpose_groups(RCH // 16)
                pltpu.sync_copy(rows2, dst_h.at[pl.ds(off, RCH)])

            @pl.when(c == NCHT - 1)
            def _tail():
                pltpu.sync_copy(src_h.at[:, pl.ds(off, TAIL)],
                                stag.at[:, pl.ds(0, TAIL)])
                transpose_groups(TAIL // 16)
                pltpu.sync_copy(rows2.at[pl.ds(0, TAIL)],
                                dst_h.at[pl.ds(off, TAIL)])

            return 0

        lax.fori_loop(0, KMAX, k_body, 0)


def _bpr_body(user_h, item_i_h, item_j_h, U_h, I_h, out_h,
              uidx, iidx, jidx, urows, iirows, ijrows, outv, sem):
    cid = lax.axis_index("c")
    sid = lax.axis_index("s")
    wid = sid * 2 + cid
    base = wid * BPW

    pltpu.sync_copy(user_h.at[wid], uidx)
    pltpu.sync_copy(item_i_h.at[wid], iidx)
    pltpu.sync_copy(item_j_h.at[wid], jidx)

    copies = []
    for ch in range(NCHUNK):
        dst = pl.ds(ch * CHUNK, CHUNK)
        copies.append(pltpu.async_copy(U_h.at[uidx.at[ch]], urows.at[dst], sem))
        copies.append(pltpu.async_copy(I_h.at[iidx.at[ch]], iirows.at[dst], sem))
        copies.append(pltpu.async_copy(I_h.at[jidx.at[ch]], ijrows.at[dst], sem))
    for c in copies:
        c.wait()

    lanes = lax.iota(jnp.int32, 16)

    def group_body(g, _):
        row = g * 16 + lanes

        def d_step(d, acc):
            col = (lanes + d) & (DIM - 1)
            u = plsc.load_gather(urows, [row, col])
            ei = plsc.load_gather(iirows, [row, col])
            ej = plsc.load_gather(ijrows, [row, col])
            return acc + u * (ei - ej)

        acc = lax.fori_loop(0, DIM, d_step, jnp.zeros((16,), jnp.float32))
        outv[pl.ds(g * 16, 16)] = acc
        return 0

    lax.fori_loop(0, BPW // 16, group_body, 0)

    pltpu.sync_copy(outv, out_h.at[pl.ds(base, BPW)])


_SC_PARAMS = pltpu.CompilerParams(
    needs_layout_passes=False, use_tc_tiling_on_sc=False
)


@jax.jit
def kernel(user, item_i, item_j, U, I):
    user3 = user.astype(jnp.int32).reshape(NW, NCHUNK, CHUNK)
    item_i3 = item_i.astype(jnp.int32).reshape(NW, NCHUNK, CHUNK)
    item_j3 = item_j.astype(jnp.int32).reshape(NW, NCHUNK, CHUNK)

    mesh = plsc.VectorSubcoreMesh(core_axis_name="c", subcore_axis_name="s")

    repack = functools.partial(
        pl.kernel,
        out_type=(
            jax.ShapeDtypeStruct((NROWS, DIM), jnp.float32),
            jax.ShapeDtypeStruct((NROWS, DIM), jnp.float32),
        ),
        mesh=mesh,
        compiler_params=_SC_PARAMS,
        scratch_types=[
            pltpu.VMEM((DIM, RCH), jnp.float32),
            pltpu.VMEM((RCH, DIM), jnp.float32),
            pltpu.SemaphoreType.DMA,
        ],
    )(_repack_body)
    Urm, Irm = repack(U.T, I.T)

    gather = functools.partial(
        pl.kernel,
        out_type=jax.ShapeDtypeStruct((BATCH,), jnp.float32),
        mesh=mesh,
        compiler_params=_SC_PARAMS,
        scratch_types=[
            pltpu.VMEM((NCHUNK, CHUNK), jnp.int32),
            pltpu.VMEM((NCHUNK, CHUNK), jnp.int32),
            pltpu.VMEM((NCHUNK, CHUNK), jnp.int32),
            pltpu.VMEM((BPW, DIM), jnp.float32),
            pltpu.VMEM((BPW, DIM), jnp.float32),
            pltpu.VMEM((BPW, DIM), jnp.float32),
            pltpu.VMEM((BPW,), jnp.float32),
            pltpu.SemaphoreType.DMA,
        ],
    )(_bpr_body)
    return gather(user3, item_i3, item_j3, Urm, Irm)


# speedup vs baseline: 1.1895x; 1.1895x over previous
import functools

import jax
import jax.numpy as jnp
from jax import lax
from jax.experimental import pallas as pl
from jax.experimental.pallas import tpu as pltpu
from jax.experimental.pallas import tpu_sc as plsc

BATCH = 16384
DIM = 32
NW = 32
BPW = BATCH // NW
NCHUNK = 4
CHUNK = BPW // NCHUNK

NROWS = 1000000
RCH = 1024
NPAN = NROWS // RCH + 1
TAIL = NROWS - (NPAN - 1) * RCH
KMAX = -(-NPAN // NW)


def _repack_body(Ut_h, It_h, Urm_h, Irm_h, stag, rows2, sem):
    cid = lax.axis_index("c")
    sid = lax.axis_index("s")
    wid = sid * 2 + cid

    lanes = lax.iota(jnp.int32, 16)

    def transpose_groups(ngrp):
        def g_body(g, _):
            row = g * 16 + lanes
            for d in range(DIM):
                vals = stag[d, pl.ds(g * 16, 16)]
                col = (lanes + d) & (DIM - 1)
                plsc.store_scatter(rows2, [row, col], vals)
            return 0

        lax.fori_loop(0, ngrp, g_body, 0)

    for src_h, dst_h in ((Ut_h, Urm_h), (It_h, Irm_h)):

        def k_body(k, _):
            c = wid + k * NW
            off = c * RCH

            @pl.when(c < NPAN - 1)
            def _full():
                pltpu.sync_copy(src_h.at[:, pl.ds(off, RCH)], stag)
                transpose_groups(RCH // 16)
                pltpu.sync_copy(rows2, dst_h.at[pl.ds(off, RCH)])

            @pl.when(c == NPAN - 1)
            def _tail():
                pltpu.sync_copy(src_h.at[:, pl.ds(off, TAIL)],
                                stag.at[:, pl.ds(0, TAIL)])
                transpose_groups(TAIL // 16)
                pltpu.sync_copy(rows2.at[pl.ds(0, TAIL)],
                                dst_h.at[pl.ds(off, TAIL)])

            return 0

        lax.fori_loop(0, KMAX, k_body, 0)


def _bpr_body(user_h, item_i_h, item_j_h, ulane_h, ilane_h, jlane_h,
              U_h, I_h, out_h,
              uidx, iidx, jidx, ulane, ilane, jlane,
              urows, iirows, ijrows, outv, sem):
    cid = lax.axis_index("c")
    sid = lax.axis_index("s")
    wid = sid * 2 + cid
    base = wid * BPW

    pltpu.sync_copy(user_h.at[wid], uidx)
    pltpu.sync_copy(item_i_h.at[wid], iidx)
    pltpu.sync_copy(item_j_h.at[wid], jidx)
    pltpu.sync_copy(ulane_h.at[wid], ulane)
    pltpu.sync_copy(ilane_h.at[wid], ilane)
    pltpu.sync_copy(jlane_h.at[wid], jlane)

    copies = []
    for ch in range(NCHUNK):
        dst = pl.ds(ch * CHUNK, CHUNK)
        copies.append(pltpu.async_copy(U_h.at[uidx.at[ch]], urows.at[dst], sem))
        copies.append(pltpu.async_copy(I_h.at[iidx.at[ch]], iirows.at[dst], sem))
        copies.append(pltpu.async_copy(I_h.at[jidx.at[ch]], ijrows.at[dst], sem))
    for c in copies:
        c.wait()

    lanes = lax.iota(jnp.int32, 16)

    def group_body(g, _):
        row = g * 16 + lanes
        b16 = pl.ds(g * 16, 16)
        ul = ulane[b16]
        il = ilane[b16]
        jl = jlane[b16]

        def d_step(d, acc):
            u = plsc.load_gather(urows, [row, (ul + d) & (DIM - 1)])
            ei = plsc.load_gather(iirows, [row, (il + d) & (DIM - 1)])
            ej = plsc.load_gather(ijrows, [row, (jl + d) & (DIM - 1)])
            return acc + u * (ei - ej)

        acc = lax.fori_loop(0, DIM, d_step, jnp.zeros((16,), jnp.float32))
        outv[pl.ds(g * 16, 16)] = acc
        return 0

    lax.fori_loop(0, BPW // 16, group_body, 0)

    pltpu.sync_copy(outv, out_h.at[pl.ds(base, BPW)])


_SC_PARAMS = pltpu.CompilerParams(
    needs_layout_passes=False, use_tc_tiling_on_sc=False
)


@jax.jit
def kernel(user, item_i, item_j, U, I):
    user = user.astype(jnp.int32)
    item_i = item_i.astype(jnp.int32)
    item_j = item_j.astype(jnp.int32)
    user3 = user.reshape(NW, NCHUNK, CHUNK)
    item_i3 = item_i.reshape(NW, NCHUNK, CHUNK)
    item_j3 = item_j.reshape(NW, NCHUNK, CHUNK)
    ulane2 = (user & 15).reshape(NW, BPW)
    ilane2 = (item_i & 15).reshape(NW, BPW)
    jlane2 = (item_j & 15).reshape(NW, BPW)

    mesh = plsc.VectorSubcoreMesh(core_axis_name="c", subcore_axis_name="s")

    repack = functools.partial(
        pl.kernel,
        out_type=(
            jax.ShapeDtypeStruct((NROWS, DIM), jnp.float32),
            jax.ShapeDtypeStruct((NROWS, DIM), jnp.float32),
        ),
        mesh=mesh,
        compiler_params=_SC_PARAMS,
        scratch_types=[
            pltpu.VMEM((DIM, RCH), jnp.float32),
            pltpu.VMEM((RCH, DIM), jnp.float32),
            pltpu.SemaphoreType.DMA,
        ],
    )(_repack_body)
    Urm, Irm = repack(U.T, I.T)

    gather = functools.partial(
        pl.kernel,
        out_type=jax.ShapeDtypeStruct((BATCH,), jnp.float32),
        mesh=mesh,
        compiler_params=_SC_PARAMS,
        scratch_types=[
            pltpu.VMEM((NCHUNK, CHUNK), jnp.int32),
            pltpu.VMEM((NCHUNK, CHUNK), jnp.int32),
            pltpu.VMEM((NCHUNK, CHUNK), jnp.int32),
            pltpu.VMEM((BPW,), jnp.int32),
            pltpu.VMEM((BPW,), jnp.int32),
            pltpu.VMEM((BPW,), jnp.int32),
            pltpu.VMEM((BPW, DIM), jnp.float32),
            pltpu.VMEM((BPW, DIM), jnp.float32),
            pltpu.VMEM((BPW, DIM), jnp.float32),
            pltpu.VMEM((BPW,), jnp.float32),
            pltpu.SemaphoreType.DMA,
        ],
    )(_bpr_body)
    return gather(user3, item_i3, item_j3, ulane2, ilane2, jlane2, Urm, Irm)
